# Initial kernel scaffold; baseline (speedup 1.0000x reference)
import jax
import jax.numpy as jnp
from jax.experimental import pallas as pl


def kernel(time, available_time, temperature, geopotential):
    raise NotImplementedError("write your pallas kernel here")


# speedup vs baseline: 1.6051x; 1.6051x over previous
import functools

import jax
import jax.numpy as jnp
from jax import lax
from jax.experimental import pallas as pl
from jax.experimental.pallas import tpu as pltpu
from jax.experimental.pallas import tpu_sc as plsc

_L = 16


def _interp_round_index(t, av_ref, T):
    lanes = lax.iota(jnp.int32, _L)
    ones = jnp.ones((_L,), jnp.int32)
    zeros = jnp.zeros((_L,), jnp.int32)
    cnt = jnp.int32(0)
    for c in range(T // _L):
        xs = av_ref[pl.ds(c * _L, _L)]
        cnt = cnt + lax.reduce_sum_p.bind(
            jnp.where(xs <= t, ones, zeros), axes=(0,))
    j = jnp.clip(cnt - 1, 0, T - 2)
    j_vec = jnp.full((_L,), j, jnp.int32)
    xj = plsc.load_gather(av_ref, [j_vec])
    xj1 = plsc.load_gather(av_ref, [j_vec + 1])
    jf = j_vec.astype(jnp.float32)
    approx = jf + (t - xj) / (xj1 - xj)
    approx = jnp.clip(approx, 0.0, float(T - 1))
    rcnt = jnp.int32(0)
    ties_even = jnp.int32(0)
    for c in range(T // _L):
        k = lanes + c * _L
        h = k.astype(jnp.float32) + 0.5
        rcnt = rcnt + lax.reduce_sum_p.bind(
            jnp.where(approx >= h, ones, zeros), axes=(0,))
        tie = (approx == h) & ((k & 1) == 0)
        ties_even = ties_even + lax.reduce_sum_p.bind(
            jnp.where(tie, ones, zeros), axes=(0,))
    return rcnt - ties_even


def _make_sc_kernel(T, H, W, dtype):
    rows = H // 32

    mesh = plsc.VectorSubcoreMesh(core_axis_name="c", subcore_axis_name="s")

    @functools.partial(
        pl.kernel,
        mesh=mesh,
        compiler_params=pltpu.CompilerParams(needs_layout_passes=False),
        out_type=(
            jax.ShapeDtypeStruct((H, W), dtype),
            jax.ShapeDtypeStruct((H, W), dtype),
        ),
        scratch_types=[
            pltpu.VMEM((_L,), jnp.float32),
            pltpu.VMEM((T,), jnp.float32),
            pltpu.VMEM((rows, W), dtype),
        ],
    )
    def sc_slice(time_hbm, av_hbm, temp_hbm, geo_hbm, out_t_hbm, out_g_hbm,
                 t_v, av_v, buf):
        cid = lax.axis_index("c")
        sid = lax.axis_index("s")
        pltpu.sync_copy(time_hbm, t_v)
        pltpu.sync_copy(av_hbm, av_v)
        idx = _interp_round_index(t_v[...], av_v, T)
        wid = sid * 2 + cid
        base = wid * rows
        pltpu.sync_copy(temp_hbm.at[idx, pl.ds(base, rows)], buf)
        pltpu.sync_copy(buf, out_t_hbm.at[pl.ds(base, rows)])
        pltpu.sync_copy(geo_hbm.at[idx, pl.ds(base, rows)], buf)
        pltpu.sync_copy(buf, out_g_hbm.at[pl.ds(base, rows)])

    return sc_slice


def kernel(time, available_time, temperature, geopotential):
    T = available_time.shape[0]
    H, W = temperature.shape[1], temperature.shape[2]
    t16 = jnp.broadcast_to(time.astype(jnp.float32), (_L,))
    sc = _make_sc_kernel(T, H, W, temperature.dtype)
    out_t, out_g = sc(t16, available_time.astype(jnp.float32),
                      temperature, geopotential)
    return (out_t, out_g)
